# baseline (speedup 1.0000x reference)
import jax
import jax.numpy as jnp
from jax.experimental import pallas as pl
from jax.experimental.pallas import tpu as pltpu

_THR = 0.005
_EPS = 1e-8
_D = 768


def _fused_body(x_ref, keys_ref, values_ref, init_ref, o_ref, kn_ref, vb_ref):
    @pl.when(pl.program_id(0) == 0)
    def _():
        k = keys_ref[...]
        kn = k / jnp.maximum(
            jnp.sqrt(jnp.sum(k * k, axis=-1, keepdims=True)), _EPS)
        kn_ref[...] = kn
        vb_ref[:, :_D] = values_ref[...]
        vb_ref[:, _D:] = jnp.ones_like(vb_ref[:, _D:])

    Bb = o_ref.shape[0]
    H = Bb // 2
    for h in (0, 1):
        rows = slice(h * H, (h + 1) * H)
        e = jnp.exp(x_ref[rows, :])
        rn = jax.lax.rsqrt(jnp.sum(e * e, axis=-1, keepdims=True))

        u = jax.lax.dot_general(
            e, kn_ref[...], (((1,), (1,)), ((), ())),
            preferred_element_type=jnp.float32)
        sim = u * rn

        se = jnp.where(sim > _THR, jnp.exp(sim), 0.0)

        ret = jnp.dot(se, vb_ref[...],
                      preferred_element_type=jnp.float32)
        ssum = ret[:, _D:_D + 1]
        retrieved = ret[:, :_D] / ssum
        o_ref[rows, :] = jnp.where(ssum > 0.0, retrieved, init_ref[...])


def kernel(output, keys, values, init_prompt):
    B, C = output.shape
    K, D = values.shape
    Bb = 1024

    initp = init_prompt.reshape(1, D)

    return pl.pallas_call(
        _fused_body,
        grid=(B // Bb,),
        in_specs=[
            pl.BlockSpec((Bb, C), lambda i: (i, 0)),
            pl.BlockSpec((K, C), lambda i: (0, 0)),
            pl.BlockSpec((K, D), lambda i: (0, 0)),
            pl.BlockSpec((1, D), lambda i: (0, 0)),
        ],
        out_specs=pl.BlockSpec((Bb, D), lambda i: (i, 0)),
        out_shape=jax.ShapeDtypeStruct((B, D), jnp.float32),
        scratch_shapes=[pltpu.VMEM((K, C), jnp.float32),
                        pltpu.VMEM((K, D + 128), jnp.float32)],
    )(output, keys, values, initp)


# speedup vs baseline: 1.0053x; 1.0053x over previous
import jax
import jax.numpy as jnp
from jax.experimental import pallas as pl
from jax.experimental.pallas import tpu as pltpu

_THR = 0.005
_EPS = 1e-8
_D = 768


def _fused_body(x_ref, keys_ref, values_ref, init_ref, o_ref, kn_ref, vb_ref):
    @pl.when(pl.program_id(0) == 0)
    def _():
        k = keys_ref[...]
        kn = k / jnp.maximum(
            jnp.sqrt(jnp.sum(k * k, axis=-1, keepdims=True)), _EPS)
        kn_ref[...] = kn.astype(jnp.bfloat16)
        vb_ref[:, :_D] = values_ref[...].astype(jnp.bfloat16)
        vb_ref[:, _D:] = jnp.ones_like(vb_ref[:, _D:])

    e = jnp.exp(x_ref[...]).astype(jnp.bfloat16)
    e32 = e.astype(jnp.float32)
    rn = jax.lax.rsqrt(jnp.sum(e32 * e32, axis=-1, keepdims=True))

    u = jax.lax.dot_general(
        e, kn_ref[...], (((1,), (1,)), ((), ())),
        preferred_element_type=jnp.float32)
    sim = u * rn

    se = jnp.where(sim > _THR, jnp.exp(sim),
                   0.0).astype(jnp.bfloat16)

    ret = jnp.dot(se, vb_ref[...],
                  preferred_element_type=jnp.float32)
    ssum = ret[:, _D:_D + 1]
    retrieved = ret[:, :_D] / ssum
    o_ref[...] = jnp.where(ssum > 0.0, retrieved, init_ref[...])


def kernel(output, keys, values, init_prompt):
    B, C = output.shape
    K, D = values.shape
    Bb = 1024

    initp = init_prompt.reshape(1, D)

    return pl.pallas_call(
        _fused_body,
        grid=(B // Bb,),
        in_specs=[
            pl.BlockSpec((Bb, C), lambda i: (i, 0)),
            pl.BlockSpec((K, C), lambda i: (0, 0)),
            pl.BlockSpec((K, D), lambda i: (0, 0)),
            pl.BlockSpec((1, D), lambda i: (0, 0)),
        ],
        out_specs=pl.BlockSpec((Bb, D), lambda i: (i, 0)),
        out_shape=jax.ShapeDtypeStruct((B, D), jnp.float32),
        scratch_shapes=[pltpu.VMEM((K, C), jnp.bfloat16),
                        pltpu.VMEM((K, D + 128), jnp.bfloat16)],
    )(output, keys, values, initp)


# speedup vs baseline: 1.0154x; 1.0100x over previous
import jax
import jax.numpy as jnp
from jax.experimental import pallas as pl
from jax.experimental.pallas import tpu as pltpu

_THR = 0.005
_EPS = 1e-8
_D = 768
_C = 1000
_K = 1000
_CP = 1024
_KP = 1024


def _fused_body(x_ref, keys_ref, values_ref, init_ref, o_ref,
                kn_ref, vb_ref, e_ref):
    @pl.when(pl.program_id(0) == 0)
    def _():
        k = keys_ref[...]
        kn = k / jnp.maximum(
            jnp.sqrt(jnp.sum(k * k, axis=-1, keepdims=True)), _EPS)
        kn_ref[...] = jnp.zeros_like(kn_ref)
        kn_ref[:_K, :_C] = kn
        vb_ref[...] = jnp.zeros_like(vb_ref)
        vb_ref[:_K, :_D] = values_ref[...]
        vb_ref[:_K, _D:] = jnp.ones_like(vb_ref[:_K, _D:])
        e_ref[:, _C:] = jnp.zeros_like(e_ref[:, _C:])

    e_ref[:, :_C] = jnp.exp(x_ref[...])
    e = e_ref[...]
    rn = jax.lax.rsqrt(jnp.sum(e * e, axis=-1, keepdims=True))

    u = jax.lax.dot_general(
        e, kn_ref[...], (((1,), (1,)), ((), ())),
        preferred_element_type=jnp.float32)
    sim = u * rn

    se = jnp.where(sim > _THR, jnp.exp(sim), 0.0)

    ret = jnp.dot(se, vb_ref[...],
                  preferred_element_type=jnp.float32)
    ssum = ret[:, _D:_D + 1]
    retrieved = ret[:, :_D] / ssum
    o_ref[...] = jnp.where(ssum > 0.0, retrieved, init_ref[...])


def kernel(output, keys, values, init_prompt):
    B, C = output.shape
    K, D = values.shape
    Bb = 1024

    initp = init_prompt.reshape(1, D)

    return pl.pallas_call(
        _fused_body,
        grid=(B // Bb,),
        in_specs=[
            pl.BlockSpec((Bb, C), lambda i: (i, 0)),
            pl.BlockSpec((K, C), lambda i: (0, 0)),
            pl.BlockSpec((K, D), lambda i: (0, 0)),
            pl.BlockSpec((1, D), lambda i: (0, 0)),
        ],
        out_specs=pl.BlockSpec((Bb, D), lambda i: (i, 0)),
        out_shape=jax.ShapeDtypeStruct((B, D), jnp.float32),
        scratch_shapes=[pltpu.VMEM((_KP, _CP), jnp.float32),
                        pltpu.VMEM((_KP, D + 128), jnp.float32),
                        pltpu.VMEM((Bb, _CP), jnp.float32)],
    )(output, keys, values, initp)


# speedup vs baseline: 1.0294x; 1.0139x over previous
import jax
import jax.numpy as jnp
from jax.experimental import pallas as pl
from jax.experimental.pallas import tpu as pltpu

_THR = 0.005
_EPS = 1e-8
_D = 768


def _fused_body(x_ref, keys_ref, values_ref, init_ref, o_ref, kn_ref, vb_ref):
    @pl.when(pl.program_id(0) == 0)
    def _():
        k = keys_ref[...]
        kn = k / jnp.maximum(
            jnp.sqrt(jnp.sum(k * k, axis=-1, keepdims=True)), _EPS)
        kn_ref[...] = kn
        vb_ref[:, :_D] = values_ref[...]
        vb_ref[:, _D:] = jnp.ones_like(vb_ref[:, _D:])

    e = jnp.exp(x_ref[...])
    rn = jax.lax.rsqrt(jnp.sum(e * e, axis=-1, keepdims=True))

    u = jax.lax.dot_general(
        e, kn_ref[...], (((1,), (1,)), ((), ())),
        preferred_element_type=jnp.float32)
    sim = u * rn

    se = jnp.where(sim > _THR, jnp.exp(sim), 0.0)

    ret = jnp.dot(se, vb_ref[...],
                  preferred_element_type=jnp.float32)
    ssum = ret[:, _D:_D + 1]
    retrieved = ret[:, :_D] / ssum
    o_ref[...] = jnp.where(ssum > 0.0, retrieved, init_ref[...])


def kernel(output, keys, values, init_prompt):
    B, C = output.shape
    K, D = values.shape
    Bb = 1024

    initp = init_prompt.reshape(1, D)

    return pl.pallas_call(
        _fused_body,
        grid=(B // Bb,),
        in_specs=[
            pl.BlockSpec((Bb, C), lambda i: (i, 0)),
            pl.BlockSpec((K, C), lambda i: (0, 0)),
            pl.BlockSpec((K, D), lambda i: (0, 0)),
            pl.BlockSpec((1, D), lambda i: (0, 0)),
        ],
        out_specs=pl.BlockSpec((Bb, D), lambda i: (i, 0)),
        out_shape=jax.ShapeDtypeStruct((B, D), jnp.float32),
        scratch_shapes=[pltpu.VMEM((K, C), jnp.float32),
                        pltpu.VMEM((K, D + 128), jnp.float32)],
    )(output, keys, values, initp)
